# Initial kernel scaffold; baseline (speedup 1.0000x reference)
import functools

import jax
import jax.numpy as jnp
from jax import lax
from jax.experimental import pallas as pl
from jax.experimental.pallas import tpu as pltpu
from jax.experimental.pallas import tpu_sc as plsc

E = 320000
N = 10000
D = 128

NC = 2
NS = 16
NW = NC * NS
LANES = 16
CHUNK = E // NW
NPAD = 10240
NSL = NPAD // NS
NROW = N // NS
W = 128
NFULL = CHUNK // W
TAIL = CHUNK - NFULL * W
NEG = -3.0e38
EPS = 1e-13

_LANE = jnp.arange(LANES, dtype=jnp.int32)

_mesh = plsc.VectorSubcoreMesh(
    core_axis_name="c", subcore_axis_name="s", num_cores=NC, num_subcores=NS)


def _take(v, idx):
  return jnp.take(v, idx, mode="promise_in_bounds")


def _seg_scan(vals, ix, op):
  for sh in (1, 2, 4, 8):
    src = jnp.maximum(_LANE - sh, 0)
    sv = _take(vals, src)
    si = _take(ix, src)
    same = (_LANE >= sh) & (si == ix)
    vals = jnp.where(same, op(vals, sv), vals)
  return vals


def _last_of_run(ix):
  nxt = _take(ix, jnp.minimum(_LANE + 1, LANES - 1))
  return (_LANE == LANES - 1) | (ix != nxt)


_BK = 2560
_GRID1 = E // _BK


def _k1_body(x_ref, wg_ref, bg_ref, wm_ref, bm_ref, gate_ref, msg_ref):
  x = x_ref[...]
  gate_ref[...] = (
      jnp.dot(x, wg_ref[...], preferred_element_type=jnp.float32)
      + bg_ref[0, 0])
  msg_ref[...] = (
      jnp.dot(x, wm_ref[...], preferred_element_type=jnp.float32)
      + bm_ref[...])


def _k1(x, Wg, bg2, Wm, bm2):
  return pl.pallas_call(
      _k1_body,
      grid=(_GRID1,),
      in_specs=[
          pl.BlockSpec((_BK, D), lambda i: (i, 0)),
          pl.BlockSpec((D, 1), lambda i: (0, 0)),
          pl.BlockSpec((1, 1), lambda i: (0, 0)),
          pl.BlockSpec((D, D), lambda i: (0, 0)),
          pl.BlockSpec((1, D), lambda i: (0, 0)),
      ],
      out_specs=[
          pl.BlockSpec((_BK, 1), lambda i: (i, 0)),
          pl.BlockSpec((_BK, D), lambda i: (i, 0)),
      ],
      out_shape=[
          jax.ShapeDtypeStruct((E, 1), jnp.float32),
          jax.ShapeDtypeStruct((E, D), jnp.float32),
      ],
  )(x, Wg, bg2, Wm, bm2)


def _k2a_body(gate_hbm, idx_hbm, mpart_hbm, g_buf, i_buf, m_tab, red, out_sl,
              shared_m):
  c = lax.axis_index("c")
  s = lax.axis_index("s")
  wid = c * NS + s
  base = wid * CHUNK
  pltpu.sync_copy(gate_hbm.at[pl.ds(base, CHUNK)], g_buf)
  pltpu.sync_copy(idx_hbm.at[pl.ds(base, CHUNK)], i_buf)

  def init(i, _):
    m_tab[pl.ds(i * LANES, LANES)] = jnp.full((LANES,), NEG, jnp.float32)
    return 0
  lax.fori_loop(0, NPAD // LANES, init, 0)

  def step(i, _):
    g = g_buf[pl.ds(i * LANES, LANES)]
    ix = i_buf[pl.ds(i * LANES, LANES)]
    g = _seg_scan(g, ix, jnp.maximum)
    last = _last_of_run(ix)
    cur = plsc.load_gather(m_tab, [ix], mask=last)
    plsc.store_scatter(m_tab, [ix], jnp.maximum(cur, g), mask=last)
    return 0
  lax.fori_loop(0, CHUNK // LANES, step, 0)

  pltpu.sync_copy(m_tab, shared_m.at[s])
  plsc.subcore_barrier()
  pltpu.sync_copy(shared_m.at[:, pl.ds(s * NSL, NSL)], red)

  def red_step(j, _):
    acc = red[0, pl.ds(j * LANES, LANES)]
    for k in range(1, NS):
      acc = jnp.maximum(acc, red[k, pl.ds(j * LANES, LANES)])
    out_sl[pl.ds(j * LANES, LANES)] = acc
    return 0
  lax.fori_loop(0, NSL // LANES, red_step, 0)
  pltpu.sync_copy(out_sl, mpart_hbm.at[pl.ds(c * NPAD + s * NSL, NSL)])


def _k2a(gate, index):
  return pl.kernel(
      _k2a_body,
      out_type=jax.ShapeDtypeStruct((NC * NPAD,), jnp.float32),
      mesh=_mesh,
      scratch_types=[
          pltpu.VMEM((CHUNK,), jnp.float32),
          pltpu.VMEM((CHUNK,), jnp.int32),
          pltpu.VMEM((NPAD,), jnp.float32),
          pltpu.VMEM((NS, NSL), jnp.float32),
          pltpu.VMEM((NSL,), jnp.float32),
          pltpu.VMEM_SHARED((NS, NPAD), jnp.float32),
      ],
  )(gate, index)


def _k2c_body(gate_hbm, idx_hbm, w_hbm, mpart_hbm, t_hbm, spart_hbm,
              g_buf, i_buf, w_buf, t_buf, m_tab, s_tab, red, out_sl, shared_s):
  c = lax.axis_index("c")
  s = lax.axis_index("s")
  wid = c * NS + s
  base = wid * CHUNK
  pltpu.sync_copy(gate_hbm.at[pl.ds(base, CHUNK)], g_buf)
  pltpu.sync_copy(idx_hbm.at[pl.ds(base, CHUNK)], i_buf)
  pltpu.sync_copy(w_hbm.at[pl.ds(base, CHUNK)], w_buf)
  pltpu.sync_copy(mpart_hbm.at[pl.ds(0, NPAD)], m_tab)
  pltpu.sync_copy(mpart_hbm.at[pl.ds(NPAD, NPAD)], s_tab)

  def minit(i, _):
    sl = pl.ds(i * LANES, LANES)
    m_tab[sl] = jnp.maximum(m_tab[sl], s_tab[sl])
    s_tab[sl] = jnp.zeros((LANES,), jnp.float32)
    return 0
  lax.fori_loop(0, NPAD // LANES, minit, 0)

  def step(i, _):
    sl = pl.ds(i * LANES, LANES)
    g = g_buf[sl]
    ix = i_buf[sl]
    w = w_buf[sl]
    mx = plsc.load_gather(m_tab, [ix])
    t = w * jnp.exp(g - mx)
    t_buf[sl] = t
    t = _seg_scan(t, ix, lambda a, b: a + b)
    last = _last_of_run(ix)
    cur = plsc.load_gather(s_tab, [ix], mask=last)
    plsc.store_scatter(s_tab, [ix], cur + t, mask=last)
    return 0
  lax.fori_loop(0, CHUNK // LANES, step, 0)

  pltpu.sync_copy(t_buf, t_hbm.at[pl.ds(base, CHUNK)])

  pltpu.sync_copy(s_tab, shared_s.at[s])
  plsc.subcore_barrier()
  pltpu.sync_copy(shared_s.at[:, pl.ds(s * NSL, NSL)], red)

  def red_step(j, _):
    acc = red[0, pl.ds(j * LANES, LANES)]
    for k in range(1, NS):
      acc = acc + red[k, pl.ds(j * LANES, LANES)]
    out_sl[pl.ds(j * LANES, LANES)] = acc
    return 0
  lax.fori_loop(0, NSL // LANES, red_step, 0)
  pltpu.sync_copy(out_sl, spart_hbm.at[pl.ds(c * NPAD + s * NSL, NSL)])


def _k2c(gate, index, w_flat, m_part):
  return pl.kernel(
      _k2c_body,
      out_type=(
          jax.ShapeDtypeStruct((E,), jnp.float32),
          jax.ShapeDtypeStruct((NC * NPAD,), jnp.float32),
      ),
      mesh=_mesh,
      scratch_types=[
          pltpu.VMEM((CHUNK,), jnp.float32),
          pltpu.VMEM((CHUNK,), jnp.int32),
          pltpu.VMEM((CHUNK,), jnp.float32),
          pltpu.VMEM((CHUNK,), jnp.float32),
          pltpu.VMEM((NPAD,), jnp.float32),
          pltpu.VMEM((NPAD,), jnp.float32),
          pltpu.VMEM((NS, NSL), jnp.float32),
          pltpu.VMEM((NSL,), jnp.float32),
          pltpu.VMEM_SHARED((NS, NPAD), jnp.float32),
      ],
  )(gate, index, w_flat, m_part)


def _k3_body(msg_hbm, t_hbm, idx_hbm, spart_hbm, opart_hbm,
             s_tab, stage, ix_w, t_w, cf_w, rows, zbuf, out_tab):
  c = lax.axis_index("c")
  s = lax.axis_index("s")
  wid = c * NS + s
  base = wid * CHUNK

  pltpu.sync_copy(spart_hbm.at[pl.ds(0, NPAD)], s_tab)
  pltpu.sync_copy(spart_hbm.at[pl.ds(NPAD, NPAD)], stage)

  def sinit(i, _):
    sl = pl.ds(i * LANES, LANES)
    s_tab[sl] = s_tab[sl] + stage[sl]
    return 0
  lax.fori_loop(0, NPAD // LANES, sinit, 0)

  def zrow(i, _):
    def zcol(j, _):
      zbuf[i, pl.ds(j * LANES, LANES)] = jnp.zeros((LANES,), jnp.float32)
      return 0
    lax.fori_loop(0, D // LANES, zcol, 0)
    return 0
  lax.fori_loop(0, W, zrow, 0)
  r0 = s * NROW
  nz = NROW // W
  for z in range(nz):
    pltpu.sync_copy(zbuf, out_tab.at[pl.ds(r0 + z * W, W), :])
  rem = NROW - nz * W
  pltpu.sync_copy(zbuf.at[pl.ds(0, rem), :],
                  out_tab.at[pl.ds(r0 + nz * W, rem), :])
  plsc.subcore_barrier()

  def window(base_e, nw):
    pltpu.sync_copy(t_hbm.at[pl.ds(base_e, nw)], t_w.at[pl.ds(0, nw)])
    pltpu.sync_copy(idx_hbm.at[pl.ds(base_e, nw)], ix_w.at[pl.ds(0, nw)])
    pltpu.sync_copy(msg_hbm.at[pl.ds(base_e, nw), :], rows.at[pl.ds(0, nw), :])

    def coef_step(j, _):
      sl = pl.ds(j * LANES, LANES)
      ix = ix_w[sl]
      sv = plsc.load_gather(s_tab, [ix])
      cf_w[sl] = t_w[sl] / (sv + EPS)
      return 0
    lax.fori_loop(0, nw // LANES, coef_step, 0)

    def scale(e, _):
      cf = jnp.broadcast_to(cf_w[e], (LANES,))
      for k in range(D // LANES):
        sl = pl.ds(k * LANES, LANES)
        rows[e, sl] = rows[e, sl] * cf
      return 0
    lax.fori_loop(0, nw, scale, 0)

  def wstep(wi, _):
    window(base + wi * W, W)
    pltpu.sync_copy(rows, out_tab.at[ix_w], add=True)
    return 0
  lax.fori_loop(0, NFULL, wstep, 0)
  window(base + NFULL * W, TAIL)
  pltpu.sync_copy(rows.at[pl.ds(0, TAIL), :],
                  out_tab.at[ix_w.at[pl.ds(0, TAIL)]], add=True)

  plsc.subcore_barrier()
  pltpu.sync_copy(out_tab.at[pl.ds(r0, NROW), :],
                  opart_hbm.at[pl.ds(c * N + r0, NROW), :])


def _k3(msg, t, index, s_part):
  return pl.kernel(
      _k3_body,
      out_type=jax.ShapeDtypeStruct((NC * N, D), jnp.float32),
      mesh=_mesh,
      scratch_types=[
          pltpu.VMEM((NPAD,), jnp.float32),
          pltpu.VMEM((NPAD,), jnp.float32),
          pltpu.VMEM((W,), jnp.int32),
          pltpu.VMEM((W,), jnp.float32),
          pltpu.VMEM((W,), jnp.float32),
          pltpu.VMEM((W, D), jnp.float32),
          pltpu.VMEM((W, D), jnp.float32),
          pltpu.VMEM_SHARED((N, D), jnp.float32),
      ],
  )(msg, t, index, s_part)


_BN = 1000


def _k4_body(a_ref, b_ref, o_ref):
  o_ref[...] = a_ref[...] + b_ref[...]


def _k4(a, b):
  return pl.pallas_call(
      _k4_body,
      grid=(N // _BN,),
      in_specs=[
          pl.BlockSpec((_BN, D), lambda i: (i, 0)),
          pl.BlockSpec((_BN, D), lambda i: (i, 0)),
      ],
      out_specs=pl.BlockSpec((_BN, D), lambda i: (i, 0)),
      out_shape=jax.ShapeDtypeStruct((N, D), jnp.float32),
  )(a, b)


def kernel(x, index, weights, Wg, bg, Wm, bm):
  gate2, msg = _k1(x, Wg, bg.reshape(1, 1), Wm, bm.reshape(1, D))
  gate = gate2.reshape(E)
  w_flat = weights.reshape(E)
  m_part = _k2a(gate, index)
  t, s_part = _k2c(gate, index, w_flat, m_part)
  opart = _k3(msg, t, index, s_part)
  return _k4(opart[:N], opart[N:])


# speedup vs baseline: 6.7443x; 6.7443x over previous
import numpy as np

import jax
import jax.numpy as jnp
from jax import lax
from jax.experimental import pallas as pl
from jax.experimental.pallas import tpu as pltpu
from jax.experimental.pallas import tpu_sc as plsc

E = 320000
N = 10000
D = 128

NC = 2
NS = 16
NW = NC * NS
LANES = 16
CHUNK = E // NW
NPAD = 10240
NSL = NPAD // NS
NROW = N // NS
W = 128
NFULL = CHUNK // W
TAIL = CHUNK - NFULL * W
NEG = -3.0e38
EPS = 1e-13

def _lane():
  return lax.iota(jnp.int32, LANES)

_mesh = plsc.VectorSubcoreMesh(
    core_axis_name="c", subcore_axis_name="s", num_cores=NC, num_subcores=NS)


def _take(v, idx):
  return v.at[idx].get(mode="promise_in_bounds")


def _seg_scan(vals, ix, op):
  lane = _lane()
  for sh in (1, 2, 4, 8):
    src = jnp.maximum(lane - sh, 0)
    sv = _take(vals, src)
    si = _take(ix, src)
    same = (lane >= sh) & (si == ix)
    vals = jnp.where(same, op(vals, sv), vals)
  return vals


def _last_of_run(ix):
  lane = _lane()
  nxt = _take(ix, jnp.minimum(lane + 1, LANES - 1))
  return (lane == LANES - 1) | (ix != nxt)


_BK = 2560
_GRID1 = E // _BK


def _k1_body(x_ref, wg_ref, bg_ref, wm_ref, bm_ref, gate_ref, msg_ref):
  x = x_ref[...]
  gate_ref[...] = (
      jnp.dot(x, wg_ref[...], preferred_element_type=jnp.float32)
      + bg_ref[0, 0])
  msg_ref[...] = (
      jnp.dot(x, wm_ref[...], preferred_element_type=jnp.float32)
      + bm_ref[...])


def _k1(x, Wg, bg2, Wm, bm2):
  return pl.pallas_call(
      _k1_body,
      grid=(_GRID1,),
      in_specs=[
          pl.BlockSpec((_BK, D), lambda i: (i, 0)),
          pl.BlockSpec((D, 1), lambda i: (0, 0)),
          pl.BlockSpec((1, 1), lambda i: (0, 0)),
          pl.BlockSpec((D, D), lambda i: (0, 0)),
          pl.BlockSpec((1, D), lambda i: (0, 0)),
      ],
      out_specs=[
          pl.BlockSpec((_BK, 1), lambda i: (i, 0)),
          pl.BlockSpec((_BK, D), lambda i: (i, 0)),
      ],
      out_shape=[
          jax.ShapeDtypeStruct((E, 1), jnp.float32),
          jax.ShapeDtypeStruct((E, D), jnp.float32),
      ],
  )(x, Wg, bg2, Wm, bm2)


def _k2a_body(gate_hbm, idx_hbm, mpart_hbm, g_buf, i_buf, m_tab, red, out_sl,
              shared_m):
  c = lax.axis_index("c")
  s = lax.axis_index("s")
  wid = c * NS + s
  base = wid * CHUNK
  pltpu.sync_copy(gate_hbm.at[pl.ds(base, CHUNK)], g_buf)
  pltpu.sync_copy(idx_hbm.at[pl.ds(base, CHUNK)], i_buf)

  def init(i, _):
    m_tab[pl.ds(i * LANES, LANES)] = jnp.full((LANES,), NEG, jnp.float32)
    return 0
  lax.fori_loop(0, NPAD // LANES, init, 0)

  def step(i, _):
    g = g_buf[pl.ds(i * LANES, LANES)]
    ix = i_buf[pl.ds(i * LANES, LANES)]
    g = _seg_scan(g, ix, jnp.maximum)
    last = _last_of_run(ix)
    cur = plsc.load_gather(m_tab, [ix], mask=last)
    plsc.store_scatter(m_tab, [ix], jnp.maximum(cur, g), mask=last)
    return 0
  lax.fori_loop(0, CHUNK // LANES, step, 0)

  pltpu.sync_copy(m_tab, shared_m.at[s])
  plsc.subcore_barrier()
  pltpu.sync_copy(shared_m.at[:, pl.ds(s * NSL, NSL)], red)

  def red_step(j, _):
    acc = red[0, pl.ds(j * LANES, LANES)]
    for k in range(1, NS):
      acc = jnp.maximum(acc, red[k, pl.ds(j * LANES, LANES)])
    out_sl[pl.ds(j * LANES, LANES)] = acc
    return 0
  lax.fori_loop(0, NSL // LANES, red_step, 0)
  pltpu.sync_copy(out_sl, mpart_hbm.at[pl.ds(c * NPAD + s * NSL, NSL)])


def _k2a(gate, index):
  return pl.kernel(
      _k2a_body,
      out_type=jax.ShapeDtypeStruct((NC * NPAD,), jnp.float32),
      mesh=_mesh,
      compiler_params=pltpu.CompilerParams(needs_layout_passes=False),
      scratch_types=[
          pltpu.VMEM((CHUNK,), jnp.float32),
          pltpu.VMEM((CHUNK,), jnp.int32),
          pltpu.VMEM((NPAD,), jnp.float32),
          pltpu.VMEM((NS, NSL), jnp.float32),
          pltpu.VMEM((NSL,), jnp.float32),
          pltpu.VMEM_SHARED((NS, NPAD), jnp.float32),
      ],
  )(gate, index)


def _k2c_body(gate_hbm, idx_hbm, w_hbm, mpart_hbm, t_hbm, spart_hbm,
              g_buf, i_buf, w_buf, t_buf, m_tab, s_tab, red, out_sl, shared_s):
  c = lax.axis_index("c")
  s = lax.axis_index("s")
  wid = c * NS + s
  base = wid * CHUNK
  pltpu.sync_copy(gate_hbm.at[pl.ds(base, CHUNK)], g_buf)
  pltpu.sync_copy(idx_hbm.at[pl.ds(base, CHUNK)], i_buf)
  pltpu.sync_copy(w_hbm.at[pl.ds(base, CHUNK)], w_buf)
  pltpu.sync_copy(mpart_hbm.at[pl.ds(0, NPAD)], m_tab)
  pltpu.sync_copy(mpart_hbm.at[pl.ds(NPAD, NPAD)], s_tab)

  def minit(i, _):
    sl = pl.ds(i * LANES, LANES)
    m_tab[sl] = jnp.maximum(m_tab[sl], s_tab[sl])
    s_tab[sl] = jnp.zeros((LANES,), jnp.float32)
    return 0
  lax.fori_loop(0, NPAD // LANES, minit, 0)

  def step(i, _):
    sl = pl.ds(i * LANES, LANES)
    g = g_buf[sl]
    ix = i_buf[sl]
    w = w_buf[sl]
    mx = plsc.load_gather(m_tab, [ix])
    t = w * jnp.exp(g - mx)
    t_buf[sl] = t
    t = _seg_scan(t, ix, lambda a, b: a + b)
    last = _last_of_run(ix)
    cur = plsc.load_gather(s_tab, [ix], mask=last)
    plsc.store_scatter(s_tab, [ix], cur + t, mask=last)
    return 0
  lax.fori_loop(0, CHUNK // LANES, step, 0)

  pltpu.sync_copy(t_buf, t_hbm.at[pl.ds(base, CHUNK)])

  pltpu.sync_copy(s_tab, shared_s.at[s])
  plsc.subcore_barrier()
  pltpu.sync_copy(shared_s.at[:, pl.ds(s * NSL, NSL)], red)

  def red_step(j, _):
    acc = red[0, pl.ds(j * LANES, LANES)]
    for k in range(1, NS):
      acc = acc + red[k, pl.ds(j * LANES, LANES)]
    out_sl[pl.ds(j * LANES, LANES)] = acc
    return 0
  lax.fori_loop(0, NSL // LANES, red_step, 0)
  pltpu.sync_copy(out_sl, spart_hbm.at[pl.ds(c * NPAD + s * NSL, NSL)])


def _k2c(gate, index, w_flat, m_part):
  return pl.kernel(
      _k2c_body,
      out_type=(
          jax.ShapeDtypeStruct((E,), jnp.float32),
          jax.ShapeDtypeStruct((NC * NPAD,), jnp.float32),
      ),
      mesh=_mesh,
      compiler_params=pltpu.CompilerParams(needs_layout_passes=False),
      scratch_types=[
          pltpu.VMEM((CHUNK,), jnp.float32),
          pltpu.VMEM((CHUNK,), jnp.int32),
          pltpu.VMEM((CHUNK,), jnp.float32),
          pltpu.VMEM((CHUNK,), jnp.float32),
          pltpu.VMEM((NPAD,), jnp.float32),
          pltpu.VMEM((NPAD,), jnp.float32),
          pltpu.VMEM((NS, NSL), jnp.float32),
          pltpu.VMEM((NSL,), jnp.float32),
          pltpu.VMEM_SHARED((NS, NPAD), jnp.float32),
      ],
  )(gate, index, w_flat, m_part)


def _k3_body(msg_hbm, t_hbm, idx_hbm, spart_hbm, opart_hbm,
             s_tab, stage, ix_w, ix_t, t_w, cf_w, rows, out_tab):
  c = lax.axis_index("c")
  s = lax.axis_index("s")
  wid = c * NS + s
  base = wid * CHUNK

  pltpu.sync_copy(spart_hbm.at[pl.ds(0, NPAD)], s_tab)
  pltpu.sync_copy(spart_hbm.at[pl.ds(NPAD, NPAD)], stage)

  def sinit(i, _):
    sl = pl.ds(i * LANES, LANES)
    s_tab[sl] = s_tab[sl] + stage[sl]
    return 0
  lax.fori_loop(0, NPAD // LANES, sinit, 0)

  def zrow(i, _):
    def zcol(j, _):
      rows[i, pl.ds(j * LANES, LANES)] = jnp.zeros((LANES,), jnp.float32)
      return 0
    lax.fori_loop(0, D // LANES, zcol, 0)
    return 0
  lax.fori_loop(0, W, zrow, 0)
  r0 = s * 640

  def zero_rows(start, n):
    full, rem = n // W, n % W
    for z in range(full):
      pltpu.sync_copy(rows, out_tab.at[pl.ds(start + z * W, W), :])
    if rem:
      pltpu.sync_copy(rows.at[pl.ds(0, rem), :],
                      out_tab.at[pl.ds(start + full * W, rem), :])

  zero_rows(r0, 400)

  @pl.when(s < NS - 1)
  def _():
    zero_rows(r0 + 400, 240)
  plsc.subcore_barrier()

  def window(base_e, nw):
    pltpu.sync_copy(t_hbm.at[pl.ds(base_e, nw)], t_w.at[pl.ds(0, nw)])
    pltpu.sync_copy(idx_hbm.at[pl.ds(base_e, nw)], ix_w.at[pl.ds(0, nw)])
    pltpu.sync_copy(msg_hbm.at[pl.ds(base_e, nw), :], rows.at[pl.ds(0, nw), :])

    def coef_step(j, _):
      sl = pl.ds(j * LANES, LANES)
      ix = ix_w[sl]
      sv = plsc.load_gather(s_tab, [ix])
      cf_w[sl] = t_w[sl] / (sv + EPS)
      return 0
    lax.fori_loop(0, nw // LANES, coef_step, 0)

    def scale(e, _):
      cf = plsc.load_gather(cf_w, [jnp.broadcast_to(e, (LANES,))])
      for k in range(D // LANES):
        sl = pl.ds(k * LANES, LANES)
        rows[e, sl] = rows[e, sl] * cf
      return 0
    lax.fori_loop(0, nw, scale, 0)

  def wstep(wi, _):
    window(base + wi * W, W)
    pltpu.sync_copy(rows, out_tab.at[ix_w], add=True)
    return 0
  lax.fori_loop(0, NFULL, wstep, 0)
  window(base + NFULL * W, TAIL)
  ix_t[pl.ds(0, TAIL)] = ix_w[pl.ds(0, TAIL)]
  pltpu.sync_copy(rows.at[pl.ds(0, TAIL), :], out_tab.at[ix_t], add=True)

  plsc.subcore_barrier()
  pltpu.sync_copy(out_tab.at[pl.ds(r0, 400), :],
                  opart_hbm.at[pl.ds(c * N + r0, 400), :])

  @pl.when(s < NS - 1)
  def _():
    pltpu.sync_copy(out_tab.at[pl.ds(r0 + 400, 240), :],
                    opart_hbm.at[pl.ds(c * N + r0 + 400, 240), :])


def _k3(msg, t, index, s_part):
  return pl.kernel(
      _k3_body,
      out_type=jax.ShapeDtypeStruct((NC * N, D), jnp.float32),
      mesh=_mesh,
      compiler_params=pltpu.CompilerParams(needs_layout_passes=False),
      scratch_types=[
          pltpu.VMEM((NPAD,), jnp.float32),
          pltpu.VMEM((NPAD,), jnp.float32),
          pltpu.VMEM((W,), jnp.int32),
          pltpu.VMEM((TAIL,), jnp.int32),
          pltpu.VMEM((W,), jnp.float32),
          pltpu.VMEM((W,), jnp.float32),
          pltpu.VMEM((W, D), jnp.float32),
          pltpu.VMEM_SHARED((N, D), jnp.float32),
      ],
  )(msg, t, index, s_part)


_BN = 1000


def _k4_body(a_ref, b_ref, o_ref):
  o_ref[...] = a_ref[...] + b_ref[...]


def _k4(a, b):
  return pl.pallas_call(
      _k4_body,
      grid=(N // _BN,),
      in_specs=[
          pl.BlockSpec((_BN, D), lambda i: (i, 0)),
          pl.BlockSpec((_BN, D), lambda i: (i, 0)),
      ],
      out_specs=pl.BlockSpec((_BN, D), lambda i: (i, 0)),
      out_shape=jax.ShapeDtypeStruct((N, D), jnp.float32),
  )(a, b)


def kernel(x, index, weights, Wg, bg, Wm, bm):
  gate2, msg = _k1(x, Wg, bg.reshape(1, 1), Wm, bm.reshape(1, D))
  gate = gate2.reshape(E)
  w_flat = weights.reshape(E)
  m_part = _k2a(gate, index)
  t, s_part = _k2c(gate, index, w_flat, m_part)
  opart = _k3(msg, t, index, s_part)
  return _k4(opart[:N], opart[N:])


# speedup vs baseline: 8.3059x; 1.2315x over previous
import numpy as np

import jax
import jax.numpy as jnp
from jax import lax
from jax.experimental import pallas as pl
from jax.experimental.pallas import tpu as pltpu
from jax.experimental.pallas import tpu_sc as plsc

E = 320000
N = 10000
D = 128

NC = 2
NS = 16
NW = NC * NS
LANES = 16
CHUNK = E // NW
NPAD = 10240
NSL = NPAD // NS
NROW = N // NS
W = 128
NFULL = CHUNK // W
TAIL = CHUNK - NFULL * W
NEG = -3.0e38
EPS = 1e-13

def _lane():
  return lax.iota(jnp.int32, LANES)

_mesh = plsc.VectorSubcoreMesh(
    core_axis_name="c", subcore_axis_name="s", num_cores=NC, num_subcores=NS)


def _take(v, idx):
  return v.at[idx].get(mode="promise_in_bounds")


def _seg_scan(vals, ix, op):
  lane = _lane()
  for sh in (1, 2, 4, 8):
    src = jnp.maximum(lane - sh, 0)
    sv = _take(vals, src)
    si = _take(ix, src)
    same = (lane >= sh) & (si == ix)
    vals = jnp.where(same, op(vals, sv), vals)
  return vals


def _last_of_run(ix):
  lane = _lane()
  nxt = _take(ix, jnp.minimum(lane + 1, LANES - 1))
  return (lane == LANES - 1) | (ix != nxt)


_BK = 2560
_GRID1 = E // _BK


def _k1_body(x_ref, wg_ref, bg_ref, wm_ref, bm_ref, gate_ref, msg_ref):
  x = x_ref[...]
  gate_ref[...] = (
      jnp.dot(x, wg_ref[...], preferred_element_type=jnp.float32)
      + bg_ref[0, 0])
  msg_ref[...] = (
      jnp.dot(x, wm_ref[...], preferred_element_type=jnp.float32)
      + bm_ref[...])


def _k1(x, Wg, bg2, Wm, bm2):
  return pl.pallas_call(
      _k1_body,
      grid=(_GRID1,),
      in_specs=[
          pl.BlockSpec((_BK, D), lambda i: (i, 0)),
          pl.BlockSpec((D, 1), lambda i: (0, 0)),
          pl.BlockSpec((1, 1), lambda i: (0, 0)),
          pl.BlockSpec((D, D), lambda i: (0, 0)),
          pl.BlockSpec((1, D), lambda i: (0, 0)),
      ],
      out_specs=[
          pl.BlockSpec((_BK, 1), lambda i: (i, 0)),
          pl.BlockSpec((_BK, D), lambda i: (i, 0)),
      ],
      out_shape=[
          jax.ShapeDtypeStruct((E, 1), jnp.float32),
          jax.ShapeDtypeStruct((E, D), jnp.float32),
      ],
  )(x, Wg, bg2, Wm, bm2)


def _k2a_body(gate_hbm, idx_hbm, mpart_hbm, g_buf, i_buf, m_tab, red, out_sl,
              shared_m):
  c = lax.axis_index("c")
  s = lax.axis_index("s")
  wid = c * NS + s
  base = wid * CHUNK
  pltpu.sync_copy(gate_hbm.at[pl.ds(base, CHUNK)], g_buf)
  pltpu.sync_copy(idx_hbm.at[pl.ds(base, CHUNK)], i_buf)

  def init(i, _):
    m_tab[pl.ds(i * LANES, LANES)] = jnp.full((LANES,), NEG, jnp.float32)
    return 0
  lax.fori_loop(0, NPAD // LANES, init, 0)

  def step(i, _):
    g = g_buf[pl.ds(i * LANES, LANES)]
    ix = i_buf[pl.ds(i * LANES, LANES)]
    g = _seg_scan(g, ix, jnp.maximum)
    last = _last_of_run(ix)
    cur = plsc.load_gather(m_tab, [ix], mask=last)
    plsc.store_scatter(m_tab, [ix], jnp.maximum(cur, g), mask=last)
    return 0
  lax.fori_loop(0, CHUNK // LANES, step, 0)

  pltpu.sync_copy(m_tab, shared_m.at[s])
  plsc.subcore_barrier()
  pltpu.sync_copy(shared_m.at[:, pl.ds(s * NSL, NSL)], red)

  def red_step(j, _):
    acc = red[0, pl.ds(j * LANES, LANES)]
    for k in range(1, NS):
      acc = jnp.maximum(acc, red[k, pl.ds(j * LANES, LANES)])
    out_sl[pl.ds(j * LANES, LANES)] = acc
    return 0
  lax.fori_loop(0, NSL // LANES, red_step, 0)
  pltpu.sync_copy(out_sl, mpart_hbm.at[pl.ds(c * NPAD + s * NSL, NSL)])


def _k2a(gate, index):
  return pl.kernel(
      _k2a_body,
      out_type=jax.ShapeDtypeStruct((NC * NPAD,), jnp.float32),
      mesh=_mesh,
      compiler_params=pltpu.CompilerParams(needs_layout_passes=False),
      scratch_types=[
          pltpu.VMEM((CHUNK,), jnp.float32),
          pltpu.VMEM((CHUNK,), jnp.int32),
          pltpu.VMEM((NPAD,), jnp.float32),
          pltpu.VMEM((NS, NSL), jnp.float32),
          pltpu.VMEM((NSL,), jnp.float32),
          pltpu.VMEM_SHARED((NS, NPAD), jnp.float32),
      ],
  )(gate, index)


def _k2c_body(gate_hbm, idx_hbm, w_hbm, mpart_hbm, t_hbm, spart_hbm,
              g_buf, i_buf, w_buf, t_buf, m_tab, s_tab, red, out_sl, shared_s):
  c = lax.axis_index("c")
  s = lax.axis_index("s")
  wid = c * NS + s
  base = wid * CHUNK
  pltpu.sync_copy(gate_hbm.at[pl.ds(base, CHUNK)], g_buf)
  pltpu.sync_copy(idx_hbm.at[pl.ds(base, CHUNK)], i_buf)
  pltpu.sync_copy(w_hbm.at[pl.ds(base, CHUNK)], w_buf)
  pltpu.sync_copy(mpart_hbm.at[pl.ds(0, NPAD)], m_tab)
  pltpu.sync_copy(mpart_hbm.at[pl.ds(NPAD, NPAD)], s_tab)

  def minit(i, _):
    sl = pl.ds(i * LANES, LANES)
    m_tab[sl] = jnp.maximum(m_tab[sl], s_tab[sl])
    s_tab[sl] = jnp.zeros((LANES,), jnp.float32)
    return 0
  lax.fori_loop(0, NPAD // LANES, minit, 0)

  def step(i, _):
    sl = pl.ds(i * LANES, LANES)
    g = g_buf[sl]
    ix = i_buf[sl]
    w = w_buf[sl]
    mx = plsc.load_gather(m_tab, [ix])
    t = w * jnp.exp(g - mx)
    t_buf[sl] = t
    t = _seg_scan(t, ix, lambda a, b: a + b)
    last = _last_of_run(ix)
    cur = plsc.load_gather(s_tab, [ix], mask=last)
    plsc.store_scatter(s_tab, [ix], cur + t, mask=last)
    return 0
  lax.fori_loop(0, CHUNK // LANES, step, 0)

  pltpu.sync_copy(t_buf, t_hbm.at[pl.ds(base, CHUNK)])

  pltpu.sync_copy(s_tab, shared_s.at[s])
  plsc.subcore_barrier()
  pltpu.sync_copy(shared_s.at[:, pl.ds(s * NSL, NSL)], red)

  def red_step(j, _):
    acc = red[0, pl.ds(j * LANES, LANES)]
    for k in range(1, NS):
      acc = acc + red[k, pl.ds(j * LANES, LANES)]
    out_sl[pl.ds(j * LANES, LANES)] = acc
    return 0
  lax.fori_loop(0, NSL // LANES, red_step, 0)
  pltpu.sync_copy(out_sl, spart_hbm.at[pl.ds(c * NPAD + s * NSL, NSL)])


def _k2c(gate, index, w_flat, m_part):
  return pl.kernel(
      _k2c_body,
      out_type=(
          jax.ShapeDtypeStruct((E,), jnp.float32),
          jax.ShapeDtypeStruct((NC * NPAD,), jnp.float32),
      ),
      mesh=_mesh,
      compiler_params=pltpu.CompilerParams(needs_layout_passes=False),
      scratch_types=[
          pltpu.VMEM((CHUNK,), jnp.float32),
          pltpu.VMEM((CHUNK,), jnp.int32),
          pltpu.VMEM((CHUNK,), jnp.float32),
          pltpu.VMEM((CHUNK,), jnp.float32),
          pltpu.VMEM((NPAD,), jnp.float32),
          pltpu.VMEM((NPAD,), jnp.float32),
          pltpu.VMEM((NS, NSL), jnp.float32),
          pltpu.VMEM((NSL,), jnp.float32),
          pltpu.VMEM_SHARED((NS, NPAD), jnp.float32),
      ],
  )(gate, index, w_flat, m_part)


def _k3_body(msg_hbm, t_hbm, idx_hbm, spart_hbm, opart_hbm,
             s_tab, ix_w2, ix_t, t_w2, cf_w, rows2, out_tab, sem_in, sem_sc):
  c = lax.axis_index("c")
  s = lax.axis_index("s")
  wid = c * NS + s
  base = wid * CHUNK

  pltpu.sync_copy(spart_hbm.at[pl.ds(0, NPAD // D), :], s_tab)
  pltpu.sync_copy(spart_hbm.at[pl.ds(NPAD // D, NPAD // D), :],
                  rows2.at[0, pl.ds(0, NPAD // D), :])

  def sinit(i, _):
    for k in range(D // LANES):
      sl = pl.ds(k * LANES, LANES)
      s_tab[i, sl] = s_tab[i, sl] + rows2[0, i, sl]
    return 0
  lax.fori_loop(0, NPAD // D, sinit, 0)

  def zrow(i, _):
    def zcol(j, _):
      rows2[0, i, pl.ds(j * LANES, LANES)] = jnp.zeros((LANES,), jnp.float32)
      return 0
    lax.fori_loop(0, D // LANES, zcol, 0)
    return 0
  lax.fori_loop(0, W, zrow, 0)
  r0 = s * 640

  def zero_rows(start, n):
    full, rem = n // W, n % W
    for z in range(full):
      pltpu.sync_copy(rows2.at[0], out_tab.at[pl.ds(start + z * W, W), :])
    if rem:
      pltpu.sync_copy(rows2.at[0, pl.ds(0, rem), :],
                      out_tab.at[pl.ds(start + full * W, rem), :])

  zero_rows(r0, 400)

  @pl.when(s < NS - 1)
  def _():
    zero_rows(r0 + 400, 240)
  plsc.subcore_barrier()

  def in_copies(wi, b):
    e0 = base + wi * W
    return (
        (t_hbm.at[pl.ds(e0, W)], t_w2.at[b], sem_in.at[b]),
        (idx_hbm.at[pl.ds(e0, W)], ix_w2.at[b], sem_in.at[b]),
        (msg_hbm.at[pl.ds(e0, W), :], rows2.at[b], sem_in.at[b]),
    )

  def issue_in(wi, b):
    for src, dst, sem in in_copies(wi, b):
      pltpu.async_copy(src, dst, sem)

  def wait_in(wi, b):
    for src, dst, sem in in_copies(wi, b):
      pltpu.make_async_copy(src, dst, sem).wait()

  def issue_sc(b):
    pltpu.async_copy(rows2.at[b], out_tab.at[ix_w2.at[b]], sem_sc.at[b],
                     add=True)

  def wait_sc(b):
    pltpu.make_async_copy(rows2.at[b], out_tab.at[ix_w2.at[b]],
                          sem_sc.at[b]).wait()

  def compute(b, nw):
    def coef_step(j, _):
      sl = pl.ds(j * LANES, LANES)
      ix = ix_w2[b, sl]
      sv = plsc.load_gather(s_tab, [ix >> 7, ix & (D - 1)])
      cf_w[sl] = t_w2[b, sl] / (sv + EPS)
      return 0
    lax.fori_loop(0, nw // LANES, coef_step, 0)

    def scale(e4, _):
      for u in range(4):
        e = e4 * 4 + u
        cf = plsc.load_gather(cf_w, [jnp.broadcast_to(e, (LANES,))])
        for k in range(D // LANES):
          sl = pl.ds(k * LANES, LANES)
          rows2[b, e, sl] = rows2[b, e, sl] * cf
      return 0
    lax.fori_loop(0, nw // 4, scale, 0)

  issue_in(0, 0)
  issue_in(1, 1)

  def outer(j, _):
    for b in (0, 1):
      wi = j * 2 + b
      wait_in(wi, b)
      compute(b, W)
      issue_sc(b)

      @pl.when(wi >= 1)
      def _():
        wait_sc(1 - b)

        @pl.when(wi + 1 < NFULL)
        def _():
          issue_in(wi + 1, 1 - b)
    return 0
  lax.fori_loop(0, NFULL // 2, outer, 0)

  e0 = base + NFULL * W
  pltpu.sync_copy(t_hbm.at[pl.ds(e0, TAIL)], t_w2.at[0, pl.ds(0, TAIL)])
  pltpu.sync_copy(idx_hbm.at[pl.ds(e0, TAIL)], ix_w2.at[0, pl.ds(0, TAIL)])
  pltpu.sync_copy(msg_hbm.at[pl.ds(e0, TAIL), :],
                  rows2.at[0, pl.ds(0, TAIL), :])
  ix = ix_w2[0, pl.ds(0, TAIL)]
  sv = plsc.load_gather(s_tab, [ix >> 7, ix & (D - 1)])
  cf_w[pl.ds(0, TAIL)] = t_w2[0, pl.ds(0, TAIL)] / (sv + EPS)

  def tscale(e, _):
    cf = plsc.load_gather(cf_w, [jnp.broadcast_to(e, (LANES,))])
    for k in range(D // LANES):
      sl = pl.ds(k * LANES, LANES)
      rows2[0, e, sl] = rows2[0, e, sl] * cf
    return 0
  lax.fori_loop(0, TAIL, tscale, 0)
  ix_t[pl.ds(0, TAIL)] = ix_w2[0, pl.ds(0, TAIL)]
  pltpu.sync_copy(rows2.at[0, pl.ds(0, TAIL), :], out_tab.at[ix_t], add=True)
  wait_sc(1)

  plsc.subcore_barrier()
  pltpu.sync_copy(out_tab.at[pl.ds(r0, 400), :],
                  opart_hbm.at[pl.ds(c * N + r0, 400), :])

  @pl.when(s < NS - 1)
  def _():
    pltpu.sync_copy(out_tab.at[pl.ds(r0 + 400, 240), :],
                    opart_hbm.at[pl.ds(c * N + r0 + 400, 240), :])


def _k3(msg, t, index, s_part2d):
  return pl.kernel(
      _k3_body,
      out_type=jax.ShapeDtypeStruct((NC * N, D), jnp.float32),
      mesh=_mesh,
      compiler_params=pltpu.CompilerParams(needs_layout_passes=False),
      scratch_types=[
          pltpu.VMEM((NPAD // D, D), jnp.float32),
          pltpu.VMEM((2, W), jnp.int32),
          pltpu.VMEM((TAIL,), jnp.int32),
          pltpu.VMEM((2, W), jnp.float32),
          pltpu.VMEM((W,), jnp.float32),
          pltpu.VMEM((2, W, D), jnp.float32),
          pltpu.VMEM_SHARED((N, D), jnp.float32),
          pltpu.SemaphoreType.DMA((2,)),
          pltpu.SemaphoreType.DMA((2,)),
      ],
  )(msg, t, index, s_part2d)


_BN = 1000


def _k4_body(a_ref, b_ref, o_ref):
  o_ref[...] = a_ref[...] + b_ref[...]


def _k4(a, b):
  return pl.pallas_call(
      _k4_body,
      grid=(N // _BN,),
      in_specs=[
          pl.BlockSpec((_BN, D), lambda i: (i, 0)),
          pl.BlockSpec((_BN, D), lambda i: (i, 0)),
      ],
      out_specs=pl.BlockSpec((_BN, D), lambda i: (i, 0)),
      out_shape=jax.ShapeDtypeStruct((N, D), jnp.float32),
  )(a, b)


def kernel(x, index, weights, Wg, bg, Wm, bm):
  gate2, msg = _k1(x, Wg, bg.reshape(1, 1), Wm, bm.reshape(1, D))
  gate = gate2.reshape(E)
  w_flat = weights.reshape(E)
  m_part = _k2a(gate, index)
  t, s_part = _k2c(gate, index, w_flat, m_part)
  opart = _k3(msg, t, index, s_part.reshape(NC * NPAD // D, D))
  return _k4(opart[:N], opart[N:])
